# baseline (speedup 1.0000x reference)
import functools

import jax
import jax.numpy as jnp
from jax.experimental import pallas as pl
from jax.experimental.pallas import tpu as pltpu

_M = 6
_P = 30
_O = 2 * _P


_SORT_NET = ((0, 5), (1, 3), (2, 4), (1, 2), (3, 4), (0, 3), (2, 5),
             (0, 1), (2, 3), (4, 5), (1, 2), (3, 4))


def _ln(x, w, b, ones_col, eps=1e-5):
    m = jnp.dot(x, ones_col, preferred_element_type=jnp.float32,
                precision=jax.lax.Precision.HIGHEST)
    msq = jnp.dot(x * x, ones_col, preferred_element_type=jnp.float32,
                  precision=jax.lax.Precision.HIGHEST)
    v = msq - m * m
    s = jax.lax.rsqrt(v + eps)
    return (x - m) * s * w + b


def _bdot(x, w):
    return jnp.dot(x.astype(jnp.bfloat16), w,
                   preferred_element_type=jnp.float32)


def _body(actors_ref, ctrs_ref, predw_ref, predb_ref, d1w_ref, d1b_ref,
          d2w_ref, d2gw_ref, d2gb_ref, aw_ref, agw_ref, agb_ref,
          l1w_ref, l1gw_ref, l1gb_ref, l2w_ref, l2gw_ref, l2gb_ref,
          cw_ref, cb_ref, cls_out_ref, reg_out_ref, *, block):
    x = actors_ref[...]
    ctr = ctrs_ref[...]
    ctr_x = ctr[:, 0:1]
    ctr_y = ctr[:, 1:2]
    lane_par = jax.lax.broadcasted_iota(jnp.int32, (1, _O), 1) % 2
    ctr_bc = jnp.where(lane_par == 0, ctr_x, ctr_y)

    predb = predb_ref[...]
    regs = []
    dists = []
    for i in range(_M):
        p = jnp.dot(x, predw_ref[i], preferred_element_type=jnp.float32)
        reg_i = (p + predb[i:i + 1, :]) + ctr_bc
        regs.append(reg_i)
        dists.append(ctr - reg_i[:, _O - 2:_O])

    ones_col = jnp.full((x.shape[1], 1), 1.0 / x.shape[1], jnp.float32)

    dist = jnp.concatenate(dists, axis=0)
    h = _bdot(dist, d1w_ref[...]) + d1b_ref[...]
    h = jnp.maximum(h, 0.0)
    h = _bdot(h, d2w_ref[...])
    h = jnp.maximum(_ln(h, d2gw_ref[...], d2gb_ref[...], ones_col), 0.0)

    act_tiled = jnp.concatenate([x] * _M, axis=0)
    cat = jnp.concatenate([h.astype(jnp.bfloat16), act_tiled], axis=1)
    a = jnp.dot(cat, aw_ref[...], preferred_element_type=jnp.float32)
    agts = jnp.maximum(_ln(a, agw_ref[...], agb_ref[...], ones_col), 0.0)

    t = _bdot(agts, l1w_ref[...])
    t = jnp.maximum(_ln(t, l1gw_ref[...], l1gb_ref[...], ones_col), 0.0)
    t = _bdot(t, l2w_ref[...])
    t = _ln(t, l2gw_ref[...], l2gb_ref[...], ones_col)
    hfin = jnp.maximum(t + agts, 0.0)

    clsf = _bdot(hfin, cw_ref[...]) + cb_ref[0, 0]
    cls = [clsf[i * block:(i + 1) * block] for i in range(_M)]

    for i, j in _SORT_NET:
        c = cls[i] < cls[j]
        ci, cj = cls[i], cls[j]
        cls[i] = jnp.where(c, cj, ci)
        cls[j] = jnp.where(c, ci, cj)
        ri, rj = regs[i], regs[j]
        regs[i] = jnp.where(c, rj, ri)
        regs[j] = jnp.where(c, ri, rj)

    cls_out_ref[...] = jnp.concatenate(cls, axis=1)
    reg_out_ref[...] = jnp.concatenate(regs, axis=1)


def kernel(actors, actor_idcs, actor_ctrs, pred_W, pred_b, d1_W, d1_b,
           d2_W, d2_gw, d2_gb, a_W, a_gw, a_gb, lr1_W, lr1_gw, lr1_gb,
           lr2_W, lr2_gw, lr2_gb, c_W, c_b):
    del actor_idcs
    n, d = actors.shape
    block = 400 if n % 400 == 0 else n
    grid = n // block
    f32 = jnp.float32
    bf16 = jnp.bfloat16

    actors_b = actors.astype(bf16)
    pred_Wt = jnp.transpose(pred_W, (0, 2, 1)).astype(bf16)
    d1_Wt = d1_W.T.astype(bf16)
    d2_Wt = d2_W.T.astype(bf16)
    a_Wt = a_W.T.astype(bf16)
    l1_Wt = lr1_W.T.astype(bf16)
    l2_Wt = lr2_W.T.astype(bf16)
    c_Wt = c_W.T.astype(bf16)
    row = lambda v: v.reshape(1, -1)

    def full(shape):
        return pl.BlockSpec(shape, lambda i: (0,) * len(shape))

    out = pl.pallas_call(
        functools.partial(_body, block=block),
        grid=(grid,),
        in_specs=[
            pl.BlockSpec((block, d), lambda i: (i, 0)),
            pl.BlockSpec((block, 2), lambda i: (i, 0)),
            full((_M, d, _O)),
            full((_M, _O)),
            full((2, d)),
            full((1, d)),
            full((d, d)),
            full((1, d)),
            full((1, d)),
            full((2 * d, d)),
            full((1, d)),
            full((1, d)),
            full((d, d)),
            full((1, d)),
            full((1, d)),
            full((d, d)),
            full((1, d)),
            full((1, d)),
            full((d, 1)),
            full((1, 1)),
        ],
        out_specs=[
            pl.BlockSpec((block, _M), lambda i: (i, 0)),
            pl.BlockSpec((block, _M * _O), lambda i: (i, 0)),
        ],
        out_shape=[
            jax.ShapeDtypeStruct((n, _M), f32),
            jax.ShapeDtypeStruct((n, _M * _O), f32),
        ],
        compiler_params=pltpu.CompilerParams(
            dimension_semantics=("parallel",),
        ),
    )(actors_b, actor_ctrs, pred_Wt, pred_b, d1_Wt, row(d1_b), d2_Wt,
      row(d2_gw), row(d2_gb), a_Wt, row(a_gw), row(a_gb), l1_Wt,
      row(lr1_gw), row(lr1_gb), l2_Wt, row(lr2_gw), row(lr2_gb),
      c_Wt, c_b.reshape(1, 1))

    cls_sorted, reg_flat = out
    return cls_sorted, reg_flat.reshape(n, _M, _P, 2)


# speedup vs baseline: 1.8398x; 1.8398x over previous
import functools

import jax
import jax.numpy as jnp
from jax.experimental import pallas as pl
from jax.experimental.pallas import tpu as pltpu

_M = 6
_P = 30
_O = 2 * _P


_SORT_NET = ((0, 5), (1, 3), (2, 4), (1, 2), (3, 4), (0, 3), (2, 5),
             (0, 1), (2, 3), (4, 5), (1, 2), (3, 4))


def _ln(x, w, b, eps=1e-5):
    m = jnp.mean(x, axis=1, keepdims=True)
    msq = jnp.mean(x * x, axis=1, keepdims=True)
    s = jax.lax.rsqrt(msq - m * m + eps)
    return (x - m) * s * w + b


def _bdot(x, w):
    return jnp.dot(x.astype(jnp.bfloat16), w,
                   preferred_element_type=jnp.float32)


def _body(actors_ref, ctrs_ref, predw_ref, predb_ref, d1w_ref, d1b_ref,
          d2w_ref, d2gw_ref, d2gb_ref, aw_ref, agw_ref, agb_ref,
          l1w_ref, l1gw_ref, l1gb_ref, l2w_ref, l2gw_ref, l2gb_ref,
          cw_ref, cb_ref, cls_out_ref, reg_out_ref, *, block):
    x = actors_ref[...]
    ctr = ctrs_ref[...]
    ctr_x = ctr[:, 0:1]
    ctr_y = ctr[:, 1:2]
    lane_par = jax.lax.broadcasted_iota(jnp.int32, (1, _O), 1) % 2
    ctr_bc = jnp.where(lane_par == 0, ctr_x, ctr_y)

    predb = predb_ref[...]
    regs = []
    dists = []
    for i in range(_M):
        p = jnp.dot(x, predw_ref[i], preferred_element_type=jnp.float32)
        reg_i = (p + predb[i:i + 1, :]) + ctr_bc
        regs.append(reg_i)
        dists.append(ctr - reg_i[:, _O - 2:_O])

    dist = jnp.concatenate(dists, axis=0)
    h = _bdot(dist, d1w_ref[...]) + d1b_ref[...]
    h = jnp.maximum(h, 0.0)
    h = _bdot(h, d2w_ref[...])
    h = jnp.maximum(_ln(h, d2gw_ref[...], d2gb_ref[...]), 0.0)

    act_tiled = jnp.concatenate([x] * _M, axis=0)
    cat = jnp.concatenate([h.astype(jnp.bfloat16), act_tiled], axis=1)
    a = jnp.dot(cat, aw_ref[...], preferred_element_type=jnp.float32)
    agts = jnp.maximum(_ln(a, agw_ref[...], agb_ref[...]), 0.0)

    t = _bdot(agts, l1w_ref[...])
    t = jnp.maximum(_ln(t, l1gw_ref[...], l1gb_ref[...]), 0.0)
    t = _bdot(t, l2w_ref[...])
    t = _ln(t, l2gw_ref[...], l2gb_ref[...])
    hfin = jnp.maximum(t + agts, 0.0)

    clsf = _bdot(hfin, cw_ref[...]) + cb_ref[0, 0]
    cls = [clsf[i * block:(i + 1) * block] for i in range(_M)]

    for i, j in _SORT_NET:
        c = cls[i] < cls[j]
        ci, cj = cls[i], cls[j]
        cls[i] = jnp.where(c, cj, ci)
        cls[j] = jnp.where(c, ci, cj)
        ri, rj = regs[i], regs[j]
        regs[i] = jnp.where(c, rj, ri)
        regs[j] = jnp.where(c, ri, rj)

    cls_out_ref[...] = jnp.concatenate(cls, axis=1)
    reg_out_ref[...] = jnp.concatenate(regs, axis=1)


def kernel(actors, actor_idcs, actor_ctrs, pred_W, pred_b, d1_W, d1_b,
           d2_W, d2_gw, d2_gb, a_W, a_gw, a_gb, lr1_W, lr1_gw, lr1_gb,
           lr2_W, lr2_gw, lr2_gb, c_W, c_b):
    del actor_idcs
    n, d = actors.shape
    block = 400 if n % 400 == 0 else n
    grid = n // block
    f32 = jnp.float32
    bf16 = jnp.bfloat16

    actors_b = actors.astype(bf16)
    pred_Wt = jnp.transpose(pred_W, (0, 2, 1)).astype(bf16)
    d1_Wt = d1_W.T.astype(bf16)
    d2_Wt = d2_W.T.astype(bf16)
    a_Wt = a_W.T.astype(bf16)
    l1_Wt = lr1_W.T.astype(bf16)
    l2_Wt = lr2_W.T.astype(bf16)
    c_Wt = c_W.T.astype(bf16)
    row = lambda v: v.reshape(1, -1)

    def full(shape):
        return pl.BlockSpec(shape, lambda i: (0,) * len(shape))

    out = pl.pallas_call(
        functools.partial(_body, block=block),
        grid=(grid,),
        in_specs=[
            pl.BlockSpec((block, d), lambda i: (i, 0)),
            pl.BlockSpec((block, 2), lambda i: (i, 0)),
            full((_M, d, _O)),
            full((_M, _O)),
            full((2, d)),
            full((1, d)),
            full((d, d)),
            full((1, d)),
            full((1, d)),
            full((2 * d, d)),
            full((1, d)),
            full((1, d)),
            full((d, d)),
            full((1, d)),
            full((1, d)),
            full((d, d)),
            full((1, d)),
            full((1, d)),
            full((d, 1)),
            full((1, 1)),
        ],
        out_specs=[
            pl.BlockSpec((block, _M), lambda i: (i, 0)),
            pl.BlockSpec((block, _M * _O), lambda i: (i, 0)),
        ],
        out_shape=[
            jax.ShapeDtypeStruct((n, _M), f32),
            jax.ShapeDtypeStruct((n, _M * _O), f32),
        ],
        compiler_params=pltpu.CompilerParams(
            dimension_semantics=("parallel",),
        ),
    )(actors_b, actor_ctrs, pred_Wt, pred_b, d1_Wt, row(d1_b), d2_Wt,
      row(d2_gw), row(d2_gb), a_Wt, row(a_gw), row(a_gb), l1_Wt,
      row(lr1_gw), row(lr1_gb), l2_Wt, row(lr2_gw), row(lr2_gb),
      c_Wt, c_b.reshape(1, 1))

    cls_sorted, reg_flat = out
    return cls_sorted, reg_flat.reshape(n, _M, _P, 2)


# speedup vs baseline: 1.9144x; 1.0405x over previous
import functools

import jax
import jax.numpy as jnp
from jax.experimental import pallas as pl
from jax.experimental.pallas import tpu as pltpu

_M = 6
_P = 30
_O = 2 * _P


_SORT_NET = ((0, 5), (1, 3), (2, 4), (1, 2), (3, 4), (0, 3), (2, 5),
             (0, 1), (2, 3), (4, 5), (1, 2), (3, 4))


def _ln(x, w, b, eps=1e-5):
    m = jnp.mean(x, axis=1, keepdims=True)
    msq = jnp.mean(x * x, axis=1, keepdims=True)
    s = jax.lax.rsqrt(msq - m * m + eps)
    return (x - m) * s * w + b


def _bdot(x, w):
    return jnp.dot(x.astype(jnp.bfloat16), w,
                   preferred_element_type=jnp.float32)


def _body(actors_ref, ctrs_ref, predw_ref, predb_ref, d1w_ref, d1b_ref,
          d2w_ref, d2gw_ref, d2gb_ref, aw_ref, agw_ref, agb_ref,
          l1w_ref, l1gw_ref, l1gb_ref, l2w_ref, l2gw_ref, l2gb_ref,
          cw_ref, cb_ref, cls_out_ref, reg_out_ref, *, block):
    x = actors_ref[...].astype(jnp.bfloat16)
    ctr = ctrs_ref[...]
    ctr_x = ctr[:, 0:1]
    ctr_y = ctr[:, 1:2]
    lane_par = jax.lax.broadcasted_iota(jnp.int32, (1, _O), 1) % 2
    ctr_bc = jnp.where(lane_par == 0, ctr_x, ctr_y)

    predb = predb_ref[...]
    regs = []
    dists = []
    for i in range(_M):
        p = jnp.dot(x, predw_ref[i], preferred_element_type=jnp.float32)
        reg_i = (p + predb[i:i + 1, :]) + ctr_bc
        regs.append(reg_i)
        dists.append(ctr - reg_i[:, _O - 2:_O])

    dist = jnp.concatenate(dists, axis=0)
    h = _bdot(dist, d1w_ref[...]) + d1b_ref[...]
    h = jnp.maximum(h, 0.0)
    h = _bdot(h, d2w_ref[...])
    h = jnp.maximum(_ln(h, d2gw_ref[...], d2gb_ref[...]), 0.0)

    act_tiled = jnp.concatenate([x] * _M, axis=0)
    cat = jnp.concatenate([h.astype(jnp.bfloat16), act_tiled], axis=1)
    a = jnp.dot(cat, aw_ref[...], preferred_element_type=jnp.float32)
    agts = jnp.maximum(_ln(a, agw_ref[...], agb_ref[...]), 0.0)

    t = _bdot(agts, l1w_ref[...])
    t = jnp.maximum(_ln(t, l1gw_ref[...], l1gb_ref[...]), 0.0)
    t = _bdot(t, l2w_ref[...])
    t = _ln(t, l2gw_ref[...], l2gb_ref[...])
    hfin = jnp.maximum(t + agts, 0.0)

    clsf = _bdot(hfin, cw_ref[...]) + cb_ref[0, 0]
    cls = [clsf[i * block:(i + 1) * block] for i in range(_M)]

    for i, j in _SORT_NET:
        c = cls[i] < cls[j]
        ci, cj = cls[i], cls[j]
        cls[i] = jnp.where(c, cj, ci)
        cls[j] = jnp.where(c, ci, cj)
        ri, rj = regs[i], regs[j]
        regs[i] = jnp.where(c, rj, ri)
        regs[j] = jnp.where(c, ri, rj)

    cls_out_ref[...] = jnp.concatenate(cls, axis=1)
    reg_out_ref[...] = jnp.concatenate(regs, axis=1)


def kernel(actors, actor_idcs, actor_ctrs, pred_W, pred_b, d1_W, d1_b,
           d2_W, d2_gw, d2_gb, a_W, a_gw, a_gb, lr1_W, lr1_gw, lr1_gb,
           lr2_W, lr2_gw, lr2_gb, c_W, c_b):
    del actor_idcs
    n, d = actors.shape
    block = n
    for cand in (1000, 400, 200, 80, 40, 8):
        if n % cand == 0:
            block = cand
            break
    grid = n // block
    f32 = jnp.float32
    bf16 = jnp.bfloat16

    pred_Wt = jnp.transpose(pred_W, (0, 2, 1)).astype(bf16)
    d1_Wt = d1_W.T.astype(bf16)
    d2_Wt = d2_W.T.astype(bf16)
    a_Wt = a_W.T.astype(bf16)
    l1_Wt = lr1_W.T.astype(bf16)
    l2_Wt = lr2_W.T.astype(bf16)
    c_Wt = c_W.T.astype(bf16)
    row = lambda v: v.reshape(1, -1)

    def full(shape):
        return pl.BlockSpec(shape, lambda i: (0,) * len(shape))

    out = pl.pallas_call(
        functools.partial(_body, block=block),
        grid=(grid,),
        in_specs=[
            pl.BlockSpec((block, d), lambda i: (i, 0)),
            pl.BlockSpec((block, 2), lambda i: (i, 0)),
            full((_M, d, _O)),
            full((_M, _O)),
            full((2, d)),
            full((1, d)),
            full((d, d)),
            full((1, d)),
            full((1, d)),
            full((2 * d, d)),
            full((1, d)),
            full((1, d)),
            full((d, d)),
            full((1, d)),
            full((1, d)),
            full((d, d)),
            full((1, d)),
            full((1, d)),
            full((d, 1)),
            full((1, 1)),
        ],
        out_specs=[
            pl.BlockSpec((block, _M), lambda i: (i, 0)),
            pl.BlockSpec((block, _M * _O), lambda i: (i, 0)),
        ],
        out_shape=[
            jax.ShapeDtypeStruct((n, _M), f32),
            jax.ShapeDtypeStruct((n, _M * _O), f32),
        ],
        compiler_params=pltpu.CompilerParams(
            dimension_semantics=("parallel",),
        ),
    )(actors, actor_ctrs, pred_Wt, pred_b, d1_Wt, row(d1_b), d2_Wt,
      row(d2_gw), row(d2_gb), a_Wt, row(a_gw), row(a_gb), l1_Wt,
      row(lr1_gw), row(lr1_gb), l2_Wt, row(lr2_gw), row(lr2_gb),
      c_Wt, c_b.reshape(1, 1))

    cls_sorted, reg_flat = out
    return cls_sorted, reg_flat.reshape(n, _M, _P, 2)


# speedup vs baseline: 3.3692x; 1.7599x over previous
import jax
import jax.numpy as jnp
from jax.experimental import pallas as pl
from jax.experimental.pallas import tpu as pltpu

_M = 6
_P = 30
_O = 2 * _P

_SORT_NET = ((0, 5), (1, 3), (2, 4), (1, 2), (3, 4), (0, 3), (2, 5),
             (0, 1), (2, 3), (4, 5), (1, 2), (3, 4))


def _lnT(x, w, b, eps=1e-5):
    m = jnp.mean(x, axis=0, keepdims=True)
    msq = jnp.mean(x * x, axis=0, keepdims=True)
    s = jax.lax.rsqrt(msq - m * m + eps)
    return (x - m) * s * w + b


def _bdot(w, x):
    return jnp.dot(w, x.astype(jnp.bfloat16),
                   preferred_element_type=jnp.float32)


def _body(actors_ref, ctrs_ref, predw_ref, predb_ref, d1w_ref, d1b_ref,
          d2w_ref, d2gw_ref, d2gb_ref, aw_ref, agw_ref, agb_ref,
          l1w_ref, l1gw_ref, l1gb_ref, l2w_ref, l2gw_ref, l2gb_ref,
          cw_ref, cb_ref, cls_out_ref, reg_out_ref):
    xt = jnp.transpose(actors_ref[...]).astype(jnp.bfloat16)
    ctr = ctrs_ref[...]
    ctr_x = ctr[0:1, :]
    ctr_y = ctr[1:2, :]
    row_par = jax.lax.broadcasted_iota(jnp.int32, (_O, 1), 0) % 2
    ctr_bc = jnp.where(row_par == 0, ctr_x, ctr_y)

    predb = predb_ref[...]
    regs = []
    cls = []
    for i in range(_M):
        p = jnp.dot(predw_ref[i], xt, preferred_element_type=jnp.float32)
        reg_i = (p + predb[:, i:i + 1]) + ctr_bc
        regs.append(reg_i)
        dist = ctr - reg_i[_O - 2:_O, :]

        h = _bdot(d1w_ref[...], dist) + d1b_ref[...]
        h = jnp.maximum(h, 0.0)
        h = _bdot(d2w_ref[...], h)
        h = jnp.maximum(_lnT(h, d2gw_ref[...], d2gb_ref[...]), 0.0)

        cat = jnp.concatenate([h.astype(jnp.bfloat16), xt], axis=0)
        a = jnp.dot(aw_ref[...], cat, preferred_element_type=jnp.float32)
        agts = jnp.maximum(_lnT(a, agw_ref[...], agb_ref[...]), 0.0)

        t = _bdot(l1w_ref[...], agts)
        t = jnp.maximum(_lnT(t, l1gw_ref[...], l1gb_ref[...]), 0.0)
        t = _bdot(l2w_ref[...], t)
        t = _lnT(t, l2gw_ref[...], l2gb_ref[...])
        hfin = jnp.maximum(t + agts, 0.0)

        cls.append(_bdot(cw_ref[...], hfin) + cb_ref[0, 0])

    for i, j in _SORT_NET:
        c = cls[i] < cls[j]
        ci, cj = cls[i], cls[j]
        cls[i] = jnp.where(c, cj, ci)
        cls[j] = jnp.where(c, ci, cj)
        ri, rj = regs[i], regs[j]
        regs[i] = jnp.where(c, rj, ri)
        regs[j] = jnp.where(c, ri, rj)

    cls_out_ref[...] = jnp.concatenate(cls, axis=0)
    for m in range(_M):
        reg_out_ref[m] = regs[m].reshape(_P, 2, regs[m].shape[1])


def kernel(actors, actor_idcs, actor_ctrs, pred_W, pred_b, d1_W, d1_b,
           d2_W, d2_gw, d2_gb, a_W, a_gw, a_gb, lr1_W, lr1_gw, lr1_gb,
           lr2_W, lr2_gw, lr2_gb, c_W, c_b):
    del actor_idcs
    n, d = actors.shape
    block = 1024
    grid = pl.cdiv(n, block)
    f32 = jnp.float32
    bf16 = jnp.bfloat16

    col = lambda v: v.reshape(-1, 1)

    def full(shape):
        return pl.BlockSpec(shape, lambda i: (0,) * len(shape))

    out = pl.pallas_call(
        _body,
        grid=(grid,),
        in_specs=[
            pl.BlockSpec((block, d), lambda i: (i, 0)),
            pl.BlockSpec((2, block), lambda i: (0, i)),
            full((_M, _O, d)),
            full((_O, _M)),
            full((d, 2)),
            full((d, 1)),
            full((d, d)),
            full((d, 1)),
            full((d, 1)),
            full((d, 2 * d)),
            full((d, 1)),
            full((d, 1)),
            full((d, d)),
            full((d, 1)),
            full((d, 1)),
            full((d, d)),
            full((d, 1)),
            full((d, 1)),
            full((1, d)),
            full((1, 1)),
        ],
        out_specs=[
            pl.BlockSpec((_M, block), lambda i: (0, i)),
            pl.BlockSpec((_M, _P, 2, block), lambda i: (0, 0, 0, i)),
        ],
        out_shape=[
            jax.ShapeDtypeStruct((_M, n), f32),
            jax.ShapeDtypeStruct((_M, _P, 2, n), f32),
        ],
        compiler_params=pltpu.CompilerParams(
            dimension_semantics=("parallel",),
        ),
    )(actors, actor_ctrs.T, pred_W.astype(bf16), pred_b.T,
      d1_W.astype(bf16), col(d1_b), d2_W.astype(bf16), col(d2_gw),
      col(d2_gb), a_W.astype(bf16), col(a_gw), col(a_gb),
      lr1_W.astype(bf16), col(lr1_gw), col(lr1_gb), lr2_W.astype(bf16),
      col(lr2_gw), col(lr2_gb), c_W.astype(bf16), c_b.reshape(1, 1))

    cls_t, reg_t = out
    return cls_t.T, jnp.transpose(reg_t, (3, 0, 1, 2))


# speedup vs baseline: 4.4028x; 1.3068x over previous
import jax
import jax.numpy as jnp
from jax.experimental import pallas as pl
from jax.experimental.pallas import tpu as pltpu

_M = 6
_P = 30
_O = 2 * _P

_SORT_NET = ((0, 5), (1, 3), (2, 4), (1, 2), (3, 4), (0, 3), (2, 5),
             (0, 1), (2, 3), (4, 5), (1, 2), (3, 4))


def _lnT(x, w, b, eps=1e-5):
    m = jnp.mean(x, axis=0, keepdims=True)
    msq = jnp.mean(x * x, axis=0, keepdims=True)
    s = jax.lax.rsqrt(msq - m * m + eps)
    return (x - m) * s * w + b


def _bdot(w, x):
    return jnp.dot(w, x.astype(jnp.bfloat16),
                   preferred_element_type=jnp.float32)


def _body(actors_ref, ctrs_ref, predw_ref, predb_ref, d1w_ref, d1b_ref,
          d2w_ref, d2gw_ref, d2gb_ref, aw_ref, agw_ref, agb_ref,
          l1w_ref, l1gw_ref, l1gb_ref, l2w_ref, l2gw_ref, l2gb_ref,
          cw_ref, cb_ref, cls_out_ref, reg_out_ref):
    xt = jnp.transpose(actors_ref[...]).astype(jnp.bfloat16)
    ctr = ctrs_ref[...]
    ctr_x = ctr[0:1, :]
    ctr_y = ctr[1:2, :]
    row_par = jax.lax.broadcasted_iota(jnp.int32, (_O, 1), 0) % 2
    ctr_bc = jnp.where(row_par == 0, ctr_x, ctr_y)

    predb = predb_ref[...]
    regs = []
    cls = []
    for i in range(_M):
        p = jnp.dot(predw_ref[i], xt, preferred_element_type=jnp.float32)
        reg_i = (p + predb[:, i:i + 1]) + ctr_bc
        regs.append(reg_i)
        dist = ctr - reg_i[_O - 2:_O, :]

        h = _bdot(d1w_ref[...], dist) + d1b_ref[...]
        h = jnp.maximum(h, 0.0)
        h = _bdot(d2w_ref[...], h)
        h = jnp.maximum(_lnT(h, d2gw_ref[...], d2gb_ref[...]), 0.0)

        cat = jnp.concatenate([h.astype(jnp.bfloat16), xt], axis=0)
        a = jnp.dot(aw_ref[...], cat, preferred_element_type=jnp.float32)
        agts = jnp.maximum(_lnT(a, agw_ref[...], agb_ref[...]), 0.0)

        t = _bdot(l1w_ref[...], agts)
        t = jnp.maximum(_lnT(t, l1gw_ref[...], l1gb_ref[...]), 0.0)
        t = _bdot(l2w_ref[...], t)
        t = _lnT(t, l2gw_ref[...], l2gb_ref[...])
        hfin = jnp.maximum(t + agts, 0.0)

        cls.append(_bdot(cw_ref[...], hfin) + cb_ref[0, 0])

    for i, j in _SORT_NET:
        c = cls[i] < cls[j]
        ci, cj = cls[i], cls[j]
        cls[i] = jnp.where(c, cj, ci)
        cls[j] = jnp.where(c, ci, cj)
        ri, rj = regs[i], regs[j]
        regs[i] = jnp.where(c, rj, ri)
        regs[j] = jnp.where(c, ri, rj)

    cls_out_ref[...] = jnp.concatenate(cls, axis=0)
    for m in range(_M):
        reg_out_ref[m] = regs[m].reshape(_P, 2, regs[m].shape[1])


def kernel(actors, actor_idcs, actor_ctrs, pred_W, pred_b, d1_W, d1_b,
           d2_W, d2_gw, d2_gb, a_W, a_gw, a_gb, lr1_W, lr1_gw, lr1_gb,
           lr2_W, lr2_gw, lr2_gb, c_W, c_b):
    del actor_idcs
    n, d = actors.shape
    block = 2048
    grid = pl.cdiv(n, block)
    f32 = jnp.float32
    bf16 = jnp.bfloat16

    col = lambda v: v.reshape(-1, 1)

    def full(shape):
        return pl.BlockSpec(shape, lambda i: (0,) * len(shape))

    out = pl.pallas_call(
        _body,
        grid=(grid,),
        in_specs=[
            pl.BlockSpec((block, d), lambda i: (i, 0)),
            pl.BlockSpec((2, block), lambda i: (0, i)),
            full((_M, _O, d)),
            full((_O, _M)),
            full((d, 2)),
            full((d, 1)),
            full((d, d)),
            full((d, 1)),
            full((d, 1)),
            full((d, 2 * d)),
            full((d, 1)),
            full((d, 1)),
            full((d, d)),
            full((d, 1)),
            full((d, 1)),
            full((d, d)),
            full((d, 1)),
            full((d, 1)),
            full((1, d)),
            full((1, 1)),
        ],
        out_specs=[
            pl.BlockSpec((_M, block), lambda i: (0, i)),
            pl.BlockSpec((_M, _P, 2, block), lambda i: (0, 0, 0, i)),
        ],
        out_shape=[
            jax.ShapeDtypeStruct((_M, n), f32),
            jax.ShapeDtypeStruct((_M, _P, 2, n), f32),
        ],
        compiler_params=pltpu.CompilerParams(
            dimension_semantics=("parallel",),
        ),
    )(actors, actor_ctrs.T, pred_W.astype(bf16), pred_b.T,
      d1_W.astype(bf16), col(d1_b), d2_W.astype(bf16), col(d2_gw),
      col(d2_gb), a_W.astype(bf16), col(a_gw), col(a_gb),
      lr1_W.astype(bf16), col(lr1_gw), col(lr1_gb), lr2_W.astype(bf16),
      col(lr2_gw), col(lr2_gb), c_W.astype(bf16), c_b.reshape(1, 1))

    cls_t, reg_t = out
    return cls_t.T, jnp.transpose(reg_t, (3, 0, 1, 2))
